# baseline (speedup 1.0000x reference)
import functools
import numpy as np
import jax
import jax.numpy as jnp
from jax import lax
from jax.experimental import pallas as pl
from jax.experimental.pallas import tpu as pltpu
from jax.experimental.pallas import tpu_sc as plsc

V = 256
D = 512
T = 2048
P = 16
K = 128
B = 4
N = B * T
L = 16


def _make_sc_kernel():
    info = plsc.get_sparse_core_info()
    NC, NS = info.num_cores, info.num_subcores
    NW = NC * NS
    rows_per_w = N // NW
    C = 128
    n_chunks = rows_per_w // C
    Q = P
    n_q = C // Q

    mesh = plsc.VectorSubcoreMesh(core_axis_name="c", subcore_axis_name="s")

    @functools.partial(
        pl.kernel,
        mesh=mesh,
        out_type=jax.ShapeDtypeStruct((B, K, P * D), jnp.float32),
        scratch_types=[
            pltpu.VMEM((rows_per_w,), jnp.int32),
            pltpu.VMEM((2, Q, D), jnp.float32),
            pltpu.VMEM((2, Q, D), jnp.float32),
            pltpu.VMEM((C // P, P * D), jnp.float32),
            pltpu.SemaphoreType.DMA,
            pltpu.SemaphoreType.DMA,
            pltpu.SemaphoreType.DMA,
            pltpu.SemaphoreType.DMA,
            pltpu.SemaphoreType.DMA,
        ],
    )
    def k(idx_hbm, emb_hbm, pos_hbm, out_hbm,
          idx_v, bufA, bufP, bufB, sG0, sG1, sP0, sP1, sW):
        semG = (sG0, sG1)
        semP = (sP0, sP1)
        cid = lax.axis_index("c")
        sid = lax.axis_index("s")
        wid = sid * NC + cid
        base = wid * rows_per_w
        tbase = base % T
        pltpu.sync_copy(idx_hbm.at[pl.ds(base, rows_per_w)], idx_v)

        hG = [None, None]
        hP = [None, None]
        hW = [None]

        def start_quarter(c, q):
            g = (c * n_q + q) % 2
            if hG[g] is not None:
                hG[g].wait()
            if hP[g] is not None:
                hP[g].wait()
            r = c * C + q * Q
            hG[g] = pltpu.async_copy(
                emb_hbm.at[idx_v.at[pl.ds(r, Q)]], bufA.at[g], semG[g])
            hP[g] = pltpu.async_copy(
                pos_hbm.at[pl.ds(tbase + r, Q)], bufP.at[g], semP[g])

        start_quarter(0, 0)
        start_quarter(0, 1)
        for c in range(n_chunks):
            for q in range(n_q):
                g = (c * n_q + q) % 2
                hG[g].wait()
                hG[g] = None
                hP[g].wait()
                hP[g] = None
                if q == 0 and hW[0] is not None:
                    hW[0].wait()
                    hW[0] = None
                a = bufA.at[g]
                pp = bufP.at[g]

                def add_row(r, _):
                    for j in range(D // L):
                        sl = pl.ds(j * L, L)
                        bufB[q, pl.ds(r * D + j * L, L)] = a[r, sl] + pp[r, sl]
                    return _

                lax.fori_loop(0, Q, add_row, 0)
                nxt = c * n_q + q + 2
                if nxt < n_chunks * n_q:
                    start_quarter(nxt // n_q, nxt % n_q)
            r0 = base + c * C
            b_id = r0 // T
            k0 = pl.multiple_of((r0 % T) // P, 8)
            hW[0] = pltpu.async_copy(
                bufB, out_hbm.at[b_id].at[pl.ds(k0, C // P)], sW)
        hW[0].wait()

    return k


_sc_kernel = _make_sc_kernel()


def kernel(bytes, emb, pos):
    idx = bytes.reshape(N)
    return _sc_kernel(idx, emb, pos)


# speedup vs baseline: 1.5730x; 1.5730x over previous
import functools
import numpy as np
import jax
import jax.numpy as jnp
from jax import lax
from jax.experimental import pallas as pl
from jax.experimental.pallas import tpu as pltpu
from jax.experimental.pallas import tpu_sc as plsc

V = 256
D = 512
T = 2048
P = 16
K = 128
B = 4
N = B * T
L = 16


def _make_sc_kernel():
    info = plsc.get_sparse_core_info()
    NC, NS = info.num_cores, info.num_subcores
    NW = NC * NS
    rows_per_w = N // NW
    C = 128
    n_chunks = rows_per_w // C
    Q = P
    n_q = C // Q
    NBUF = 3

    mesh = plsc.VectorSubcoreMesh(core_axis_name="c", subcore_axis_name="s")

    @functools.partial(
        pl.kernel,
        mesh=mesh,
        out_type=jax.ShapeDtypeStruct((B, K, P * D), jnp.float32),
        scratch_types=[
            pltpu.VMEM((rows_per_w,), jnp.int32),
            pltpu.VMEM((NBUF, Q, D), jnp.float32),
            pltpu.VMEM((C // P, P, D), jnp.float32),
            pltpu.SemaphoreType.DMA,
            pltpu.SemaphoreType.DMA,
            pltpu.SemaphoreType.DMA,
            pltpu.SemaphoreType.DMA,
            pltpu.SemaphoreType.DMA,
        ],
    )
    def k(idx_hbm, emb_hbm, pos_hbm, out_hbm,
          idx_v, bufA, bufB, sG0, sG1, sG2, sP, sW):
        semG = (sG0, sG1, sG2)
        cid = lax.axis_index("c")
        sid = lax.axis_index("s")
        wid = sid * NC + cid
        base = wid * rows_per_w
        tbase = base % T
        pltpu.sync_copy(idx_hbm.at[pl.ds(base, rows_per_w)], idx_v)

        hG = [None] * NBUF
        hP = [None]
        hW = []

        def start_gather(c, q):
            g = (c * n_q + q) % NBUF
            if hG[g] is not None:
                hG[g].wait()
            r = c * C + q * Q
            hG[g] = pltpu.async_copy(
                emb_hbm.at[idx_v.at[pl.ds(r, Q)]], bufA.at[g], semG[g])

        def start_pos(c):
            t0 = tbase + c * C
            hP[0] = pltpu.async_copy(
                pos_hbm.at[pl.ds(t0, C)].reshape(C // P, P, D), bufB, sP)

        start_gather(0, 0)
        start_gather(0, 1)
        for c in range(n_chunks):
            for h in hW:
                h.wait()
            hW = []
            start_pos(c)
            hP[0].wait()
            for q in range(n_q):
                g = (c * n_q + q) % NBUF
                hG[g].wait()
                hG[g] = None
                nxt = c * n_q + q + 2
                if nxt < n_chunks * n_q:
                    start_gather(nxt // n_q, nxt % n_q)
                a = bufA.at[g]

                @plsc.parallel_loop(0, Q * D // L, 1, unroll=8)
                def add_elem(i):
                    rr = lax.div(i, D // L)
                    jj = lax.rem(i, D // L)
                    sl = pl.ds(jj * L, L)
                    plsc.addupdate(bufB.at[q, rr, sl], a[rr, sl])

            r0 = base + c * C
            b_id = r0 // T
            k0 = pl.multiple_of((r0 % T) // P, 8)
            for p in range(P):
                hW.append(pltpu.async_copy(
                    bufB.at[:, p, :],
                    out_hbm.at[b_id].at[pl.ds(k0, C // P),
                                        pl.ds(p * D, D)], sW))
        for h in hW:
            h.wait()

    return k


_sc_kernel = _make_sc_kernel()


def kernel(bytes, emb, pos):
    idx = bytes.reshape(N)
    return _sc_kernel(idx, emb, pos)


# speedup vs baseline: 1.6175x; 1.0282x over previous
import functools
import numpy as np
import jax
import jax.numpy as jnp
from jax import lax
from jax.experimental import pallas as pl
from jax.experimental.pallas import tpu as pltpu
from jax.experimental.pallas import tpu_sc as plsc

V = 256
D = 512
T = 2048
P = 16
K = 128
B = 4
N = B * T
L = 16


def _make_sc_kernel():
    info = plsc.get_sparse_core_info()
    NC, NS = info.num_cores, info.num_subcores
    NW = NC * NS
    rows_per_w = N // NW
    C = 128
    n_chunks = rows_per_w // C
    Q = P
    n_q = C // Q
    NBUF = 5
    LOOKAHEAD = 4

    mesh = plsc.VectorSubcoreMesh(core_axis_name="c", subcore_axis_name="s")

    @functools.partial(
        pl.kernel,
        mesh=mesh,
        out_type=jax.ShapeDtypeStruct((B, K, P * D), jnp.float32),
        scratch_types=[
            pltpu.VMEM((rows_per_w,), jnp.int32),
            pltpu.VMEM((NBUF, Q, D), jnp.float32),
            pltpu.VMEM((C // P, P, D), jnp.float32),
        ] + [pltpu.SemaphoreType.DMA] * (NBUF + n_q + 1),
    )
    def k(idx_hbm, emb_hbm, pos_hbm, out_hbm, idx_v, bufA, bufB, *sems):
        semG = sems[:NBUF]
        semP = sems[NBUF:NBUF + n_q]
        sW = sems[NBUF + n_q]
        cid = lax.axis_index("c")
        sid = lax.axis_index("s")
        wid = sid * NC + cid
        base = wid * rows_per_w
        tbase = base % T
        pltpu.sync_copy(
            idx_hbm.at[base // T].at[pl.ds(tbase, rows_per_w)], idx_v)

        hG = [None] * NBUF
        hP = [None] * n_q
        hW = []

        def start_gather(c, q):
            g = (c * n_q + q) % NBUF
            if hG[g] is not None:
                hG[g].wait()
            r = c * C + q * Q
            hG[g] = pltpu.async_copy(
                emb_hbm.at[idx_v.at[pl.ds(r, Q)]], bufA.at[g], semG[g])

        def start_pos(c):
            t0 = tbase + c * C
            for j in range(n_q):
                hP[j] = pltpu.async_copy(
                    pos_hbm.at[pl.ds(t0 + j * Q, Q)], bufB.at[j], semP[j])

        for i in range(LOOKAHEAD):
            start_gather(i // n_q, i % n_q)
        for c in range(n_chunks):
            for h in hW:
                h.wait()
            hW = []
            start_pos(c)
            for q in range(n_q):
                g = (c * n_q + q) % NBUF
                hG[g].wait()
                hG[g] = None
                hP[q].wait()
                hP[q] = None
                nxt = c * n_q + q + LOOKAHEAD
                if nxt < n_chunks * n_q:
                    start_gather(nxt // n_q, nxt % n_q)
                a = bufA.at[g]

                @plsc.parallel_loop(0, Q * D // L, 1, unroll=8)
                def add_elem(i):
                    rr = lax.div(i, D // L)
                    jj = lax.rem(i, D // L)
                    sl = pl.ds(jj * L, L)
                    plsc.addupdate(bufB.at[q, rr, sl], a[rr, sl])

            r0 = base + c * C
            b_id = r0 // T
            k0 = pl.multiple_of((r0 % T) // P, 8)
            for p in range(P):
                hW.append(pltpu.async_copy(
                    bufB.at[:, p, :],
                    out_hbm.at[b_id].at[pl.ds(k0, C // P),
                                        pl.ds(p * D, D)], sW))
        for h in hW:
            h.wait()

    return k


_sc_kernel = _make_sc_kernel()


def kernel(bytes, emb, pos):
    return _sc_kernel(bytes, emb, pos)
